# baseline (speedup 1.0000x reference)
import functools

import jax
import jax.numpy as jnp
from jax import lax
from jax.experimental import pallas as pl
from jax.experimental.pallas import tpu as pltpu
from jax.experimental.pallas import tpu_sc as plsc

_NUM_VIEWS = 6
_VOCAB = 64
_BATCH = 16384
_NC = 2
_NS = 16
_NW = _NC * _NS
_LANES = 16
_BPW = _BATCH // _NW
_OPW = _BPW * _NUM_VIEWS
_GROUPS = _BPW // _LANES


def _make_sc_gather():
    mesh = plsc.VectorSubcoreMesh(core_axis_name="c", subcore_axis_name="s")

    @functools.partial(
        pl.kernel,
        mesh=mesh,
        compiler_params=pltpu.CompilerParams(needs_layout_passes=False),
        out_type=jax.ShapeDtypeStruct((_BATCH * _NUM_VIEWS,), jnp.float32),
        scratch_types=[
            pltpu.VMEM((_BPW,), jnp.int32),
            pltpu.VMEM((_VOCAB * _NUM_VIEWS,), jnp.float32),
            pltpu.VMEM((_OPW,), jnp.float32),
            pltpu.SemaphoreType.DMA,
            pltpu.SemaphoreType.DMA,
        ],
    )
    def gather_kernel(idx_hbm, tab_hbm, out_hbm, idx_v, tab_v, out_v,
                      sem_tab, sem_idx):
        wid = lax.axis_index("s") * _NC + lax.axis_index("c")
        base = wid * _BPW
        cp_tab = pltpu.async_copy(tab_hbm, tab_v, sem_tab)
        cp_idx = pltpu.async_copy(idx_hbm.at[pl.ds(base, _BPW)], idx_v,
                                  sem_idx)
        cp_tab.wait()
        cp_idx.wait()
        lane6 = lax.iota(jnp.int32, _LANES) * _NUM_VIEWS

        for g in range(_GROUPS):
            ids = idx_v[pl.ds(g * _LANES, _LANES)]
            src = ids * _NUM_VIEWS
            dst = lane6 + g * (_LANES * _NUM_VIEWS)
            for d in range(_NUM_VIEWS):
                vals = plsc.load_gather(tab_v, [src + d])
                plsc.store_scatter(out_v, [dst + d], vals)

        pltpu.sync_copy(out_v, out_hbm.at[pl.ds(base * _NUM_VIEWS, _OPW)])

    return gather_kernel


_SC_GATHER = _make_sc_gather()


def kernel(missing_pattern, tag_table):
    idx = missing_pattern.astype(jnp.int32)
    tab = tag_table.reshape(-1).astype(jnp.float32)
    flat = _SC_GATHER(idx, tab)
    return flat.reshape(_BATCH, _NUM_VIEWS)


# speedup vs baseline: 1.1781x; 1.1781x over previous
import functools

import jax
import jax.numpy as jnp
from jax import lax
from jax.experimental import pallas as pl
from jax.experimental.pallas import tpu as pltpu
from jax.experimental.pallas import tpu_sc as plsc

_NUM_VIEWS = 6
_VOCAB = 64
_BATCH = 16384
_NC = 2
_NS = 16
_NW = _NC * _NS
_LANES = 16
_BPW = _BATCH // _NW
_GROUPS = _BPW // _LANES


def _make_sc_gather():
    mesh = plsc.VectorSubcoreMesh(core_axis_name="c", subcore_axis_name="s")

    @functools.partial(
        pl.kernel,
        mesh=mesh,
        compiler_params=pltpu.CompilerParams(needs_layout_passes=False),
        out_type=jax.ShapeDtypeStruct((_BATCH, _NUM_VIEWS), jnp.float32),
        scratch_types=[
            pltpu.VMEM((_BPW,), jnp.int32),
            pltpu.VMEM((_VOCAB, _NUM_VIEWS), jnp.float32),
            pltpu.VMEM((_BPW, _NUM_VIEWS), jnp.float32),
            pltpu.SemaphoreType.DMA,
            pltpu.SemaphoreType.DMA,
        ],
    )
    def gather_kernel(idx_hbm, tab_hbm, out_hbm, idx_v, tab_v, out_v,
                      sem_tab, sem_idx):
        wid = lax.axis_index("s") * _NC + lax.axis_index("c")
        base = wid * _BPW
        cp_tab = pltpu.async_copy(tab_hbm, tab_v, sem_tab)
        cp_idx = pltpu.async_copy(idx_hbm.at[pl.ds(base, _BPW)], idx_v,
                                  sem_idx)
        cp_tab.wait()
        cp_idx.wait()
        lane = lax.iota(jnp.int32, _LANES)

        def body(g, carry):
            ids = idx_v[pl.ds(g * _LANES, _LANES)]
            rows = lane + g * _LANES
            for d in range(_NUM_VIEWS):
                col = jnp.full((_LANES,), d, jnp.int32)
                vals = plsc.load_gather(tab_v, [ids, col])
                plsc.store_scatter(out_v, [rows, col], vals)
            return carry

        lax.fori_loop(0, _GROUPS, body, 0)
        pltpu.sync_copy(out_v, out_hbm.at[pl.ds(base, _BPW)])

    return gather_kernel


_SC_GATHER = _make_sc_gather()


def kernel(missing_pattern, tag_table):
    return _SC_GATHER(missing_pattern.astype(jnp.int32), tag_table)
